# baseline (speedup 1.0000x reference)
import jax
import jax.numpy as jnp
from jax import lax
from jax.experimental import pallas as pl
from jax.experimental.pallas import tpu as pltpu
from jax.experimental.pallas import tpu_sc as plsc

N = 10000
E = 320000
D = 128
H = 128
G = 64

NP = 10240
NC = 2
NS = 16
NW = NC * NS
CH = 128
STAGES = 5
CPS = 16
CPW = STAGES * CPS
EPW = CPW * CH
EP = NW * EPW
PAIRS = CPS // 2
AR = 10112
RPS = AR // NS

PAD_DST = N


def _dot_w(a, b):
    return lax.dot_general(a.astype(jnp.bfloat16), b.astype(jnp.bfloat16),
                           (((1,), (1,)), ((), ())),
                           preferred_element_type=jnp.float32)


_mesh = plsc.VectorSubcoreMesh(
    core_axis_name="c", subcore_axis_name="s", num_cores=NC, num_subcores=NS)


def _deg_body(dst4d, out, idxv, onesv, zbuf, accsh):
    c = lax.axis_index("c")
    s = lax.axis_index("s")
    w = c * NS + s

    def fill_ones(i, _):
        onesv[pl.ds(i * 16, 16)] = jnp.full((16,), 1.0, jnp.float32)
        return 0
    lax.fori_loop(0, CH // 16, fill_ones, 0)

    @pl.when(s == 0)
    def _():
        def zb(i, _):
            zbuf[pl.ds(i * 16, 16)] = jnp.zeros((16,), jnp.float32)
            return 0
        lax.fori_loop(0, NP // 16, zb, 0)
        pltpu.sync_copy(zbuf, accsh)

    for st in range(STAGES):
        pltpu.sync_copy(dst4d.at[w, st], idxv.at[pl.ds(st * CPS, CPS)])
    plsc.subcore_barrier()

    def body(j, _):
        pltpu.sync_copy(onesv, accsh.at[idxv.at[j]], add=True)
        return 0
    lax.fori_loop(0, CPW, body, 0)
    plsc.subcore_barrier()

    @pl.when(s == 0)
    def _():
        pltpu.sync_copy(accsh, out.at[c, 0])


_deg_call = pl.kernel(
    _deg_body,
    out_type=jax.ShapeDtypeStruct((NC, 8, NP), jnp.float32),
    mesh=_mesh,
    scratch_types=[
        pltpu.VMEM((CPW, CH), jnp.int32),
        pltpu.VMEM((CH,), jnp.float32),
        pltpu.VMEM((NP,), jnp.float32),
        pltpu.VMEM_SHARED((NP,), jnp.float32),
    ],
)


def _scat_body(y, src4d, dst4d, out, sidx0, didx0, sidx1, didx1,
               rows0, rows1, accsh, sem0, sem1, isem, xsem):
    c = lax.axis_index("c")
    s = lax.axis_index("s")
    w = c * NS + s

    pltpu.async_copy(y.at[pl.ds(s * RPS, RPS)],
                     accsh.at[pl.ds(s * RPS, RPS)], isem)

    pltpu.sync_copy(src4d.at[w, 0], sidx0)
    pltpu.sync_copy(dst4d.at[w, 0], didx0)
    pltpu.async_copy(y.at[sidx0.at[0]], rows0, sem0)
    pltpu.make_async_copy(y.at[pl.ds(s * RPS, RPS)],
                          accsh.at[pl.ds(s * RPS, RPS)], isem).wait()
    plsc.subcore_barrier()

    for st in range(STAGES):
        sidx, didx = (sidx0, didx0) if st % 2 == 0 else (sidx1, didx1)
        nsidx, ndidx = (sidx1, didx1) if st % 2 == 0 else (sidx0, didx0)
        if st < STAGES - 1:
            pltpu.async_copy(src4d.at[w, st + 1], nsidx, xsem)
            pltpu.async_copy(dst4d.at[w, st + 1], ndidx, xsem)

        def pair(p, _):
            j0 = 2 * p
            pltpu.make_async_copy(y.at[sidx.at[j0]], rows0, sem0).wait()
            pltpu.async_copy(y.at[sidx.at[j0 + 1]], rows1, sem1)
            pltpu.sync_copy(rows0, accsh.at[didx.at[j0]], add=True)
            pltpu.async_copy(y.at[sidx.at[j0 + 2]], rows0, sem0)
            pltpu.make_async_copy(y.at[sidx.at[j0 + 1]], rows1, sem1).wait()
            pltpu.sync_copy(rows1, accsh.at[didx.at[j0 + 1]], add=True)
            return 0
        lax.fori_loop(0, PAIRS - 1, pair, 0)

        jl = CPS - 2
        pltpu.make_async_copy(y.at[sidx.at[jl]], rows0, sem0).wait()
        pltpu.async_copy(y.at[sidx.at[jl + 1]], rows1, sem1)
        pltpu.sync_copy(rows0, accsh.at[didx.at[jl]], add=True)
        if st < STAGES - 1:
            pltpu.make_async_copy(src4d.at[w, st + 1], nsidx, xsem).wait()
            pltpu.make_async_copy(dst4d.at[w, st + 1], ndidx, xsem).wait()
            pltpu.async_copy(y.at[nsidx.at[0]], rows0, sem0)
        pltpu.make_async_copy(y.at[sidx.at[jl + 1]], rows1, sem1).wait()
        pltpu.sync_copy(rows1, accsh.at[didx.at[jl + 1]], add=True)

    plsc.subcore_barrier()
    pltpu.sync_copy(accsh.at[pl.ds(s * RPS, RPS)],
                    out.at[c, pl.ds(s * RPS, RPS)])


_scat_call = pl.kernel(
    _scat_body,
    out_type=jax.ShapeDtypeStruct((NC, AR, D), jnp.float32),
    mesh=_mesh,
    scratch_types=[
        pltpu.VMEM((CPS, CH), jnp.int32),
        pltpu.VMEM((CPS, CH), jnp.int32),
        pltpu.VMEM((CPS, CH), jnp.int32),
        pltpu.VMEM((CPS, CH), jnp.int32),
        pltpu.VMEM((CH, D), jnp.float32),
        pltpu.VMEM((CH, D), jnp.float32),
        pltpu.VMEM_SHARED((AR, D), jnp.float32),
        pltpu.SemaphoreType.DMA,
        pltpu.SemaphoreType.DMA,
        pltpu.SemaphoreType.DMA,
        pltpu.SemaphoreType.DMA,
    ],
)


def _prep1_body(degp, x, w1, dis_ref, y1_ref):
    dd = degp[...]
    comb = dd[0] + dd[1]
    rid = lax.broadcasted_iota(jnp.int32, (8, NP), 0)
    comb = jnp.where(rid == 0, comb, 0.0)
    ones8 = jnp.ones((8, 8), jnp.bfloat16)
    degt = lax.dot_general(comb.astype(jnp.bfloat16), ones8,
                           (((0,), (0,)), ((), ())),
                           preferred_element_type=jnp.float32)
    deg = degt[0:N, 0:1] + 1.0
    dis = lax.rsqrt(deg)
    dis_ref[...] = dis
    xw = _dot_w(x[...], w1[...])
    y1_ref[0:N, :] = xw * dis
    y1_ref[N:NP, :] = jnp.zeros((NP - N, D), jnp.float32)


_prep1_call = pl.pallas_call(
    _prep1_body,
    out_shape=[
        jax.ShapeDtypeStruct((N, 1), jnp.float32),
        jax.ShapeDtypeStruct((NP, D), jnp.float32),
    ],
)


def _mid_body(acc, y1, dis, b1, w2, y2_ref):
    comb = acc[0, 0:N, :] + acc[1, 0:N, :] - y1[0:N, :]
    h1 = jnp.maximum(comb * dis[...] + b1[...], 0.0)
    xw2 = _dot_w(h1, w2[...])
    y2_ref[0:N, :] = xw2 * dis[...]
    y2_ref[N:NP, :] = jnp.zeros((NP - N, D), jnp.float32)


_mid_call = pl.pallas_call(
    _mid_body,
    out_shape=jax.ShapeDtypeStruct((NP, D), jnp.float32),
)


def _final_body(acc, y2, dis, b2, batch, wih, bih, bhh, wlin, blin, out_ref):
    h2 = ((acc[0, 0:N, :] + acc[1, 0:N, :] - y2[0:N, :]) * dis[...]
          + b2[...])
    b = batch[...]
    gids = lax.broadcasted_iota(jnp.int32, (1, G), 1)
    oh = (b == gids).astype(jnp.float32)
    h2_hi = h2.astype(jnp.bfloat16)
    h2_lo = (h2 - h2_hi.astype(jnp.float32)).astype(jnp.bfloat16)
    ohb = oh.astype(jnp.bfloat16)
    dn = (((0,), (0,)), ((), ()))
    psum = (lax.dot_general(ohb, h2_hi, dn, preferred_element_type=jnp.float32)
            + lax.dot_general(ohb, h2_lo, dn, preferred_element_type=jnp.float32))
    ones = jnp.ones((N, 1), jnp.bfloat16)
    cnt = lax.dot_general(ohb, ones, dn, preferred_element_type=jnp.float32)
    pooled = psum / jnp.maximum(cnt, 1.0)
    gi = _dot_w(pooled, wih[...]) + bih[...]
    gh = bhh[...]
    r = jax.nn.sigmoid(gi[:, :H] + gh[:, :H])
    z = jax.nn.sigmoid(gi[:, H:2 * H] + gh[:, H:2 * H])
    nn = jnp.tanh(gi[:, 2 * H:] + r * gh[:, 2 * H:])
    g = jnp.maximum((1.0 - z) * nn, 0.0)
    mu = jnp.mean(g, axis=-1, keepdims=True)
    var = jnp.mean((g - mu) ** 2, axis=-1, keepdims=True)
    gn = (g - mu) / jnp.sqrt(var + 1e-5)
    gnb = gn.astype(jnp.bfloat16).astype(jnp.float32)
    wlb = wlin[...].astype(jnp.bfloat16).astype(jnp.float32)
    out_ref[...] = jnp.sum(gnb * wlb, axis=1, keepdims=True) + blin[...]


_final_call = pl.pallas_call(
    _final_body,
    out_shape=jax.ShapeDtypeStruct((G, 1), jnp.float32),
)


def kernel(x, edge_index, batch, W1, b1, W2, b2, W_ih, W_hh, b_ih, b_hh,
           W_lin, b_lin):
    src4d = jnp.pad(edge_index[0], (0, EP - E)).reshape(NW, STAGES, CPS, CH)
    dst4d = jnp.pad(edge_index[1], (0, EP - E),
                    constant_values=PAD_DST).reshape(NW, STAGES, CPS, CH)
    deg_out = _deg_call(dst4d)
    dis, y1 = _prep1_call(deg_out, x, W1)
    acc1 = _scat_call(y1, src4d, dst4d)
    y2 = _mid_call(acc1, y1, dis, b1.reshape(1, H), W2)
    acc2 = _scat_call(y2, src4d, dst4d)
    out = _final_call(acc2, y2, dis, b2.reshape(1, H),
                      batch.reshape(N, 1), W_ih, b_ih.reshape(1, 3 * H),
                      b_hh.reshape(1, 3 * H), W_lin, b_lin.reshape(1, 1))
    return out


# speedup vs baseline: 3.2534x; 3.2534x over previous
import jax
import jax.numpy as jnp
from jax import lax
from jax.experimental import pallas as pl
from jax.experimental.pallas import tpu as pltpu
from jax.experimental.pallas import tpu_sc as plsc

N = 10000
E = 320000
D = 128
H = 128
G = 64

NP = 10240
NC = 2
NS = 16
NW = NC * NS
CH = 128
STAGES = 5
CPS = 16
CPW = STAGES * CPS
EPW = CPW * CH
EP = NW * EPW
PAIRS = CPS // 2
AR = 10112
RPS = AR // NS

PAD_DST = N


def _dot_w(a, b):
    return lax.dot_general(a.astype(jnp.bfloat16), b.astype(jnp.bfloat16),
                           (((1,), (1,)), ((), ())),
                           preferred_element_type=jnp.float32)


_mesh = plsc.VectorSubcoreMesh(
    core_axis_name="c", subcore_axis_name="s", num_cores=NC, num_subcores=NS)


def _deg_body(dst4d, out, idxv, onesv, zbuf, accsh):
    c = lax.axis_index("c")
    s = lax.axis_index("s")
    w = c * NS + s

    def fill_ones(i, _):
        onesv[pl.ds(i * 16, 16)] = jnp.full((16,), 1.0, jnp.float32)
        return 0
    lax.fori_loop(0, CH // 16, fill_ones, 0)

    @pl.when(s == 0)
    def _():
        def zb(i, _):
            zbuf[pl.ds(i * 16, 16)] = jnp.zeros((16,), jnp.float32)
            return 0
        lax.fori_loop(0, NP // 16, zb, 0)
        pltpu.sync_copy(zbuf, accsh)

    for st in range(STAGES):
        pltpu.sync_copy(dst4d.at[w, st], idxv.at[pl.ds(st * CPS, CPS)])
    plsc.subcore_barrier()

    def body(j, _):
        pltpu.sync_copy(onesv, accsh.at[idxv.at[j]], add=True)
        return 0
    lax.fori_loop(0, CPW, body, 0)
    plsc.subcore_barrier()

    @pl.when(s == 0)
    def _():
        pltpu.sync_copy(accsh, out.at[c, 0])


_deg_call = pl.kernel(
    _deg_body,
    out_type=jax.ShapeDtypeStruct((NC, 8, NP), jnp.float32),
    mesh=_mesh,
    scratch_types=[
        pltpu.VMEM((CPW, CH), jnp.int32),
        pltpu.VMEM((CH,), jnp.float32),
        pltpu.VMEM((NP,), jnp.float32),
        pltpu.VMEM_SHARED((NP,), jnp.float32),
    ],
)


def _scat_body(y, src4d, dst4d, out, sidx0, didx0, sidx1, didx1,
               rows0, rows1, accsh, sem0, sem1, isem, xsem):
    c = lax.axis_index("c")
    s = lax.axis_index("s")
    w = c * NS + s

    pltpu.async_copy(y.at[pl.ds(s * RPS, RPS)],
                     accsh.at[pl.ds(s * RPS, RPS)], isem)

    pltpu.sync_copy(src4d.at[w, 0], sidx0)
    pltpu.sync_copy(dst4d.at[w, 0], didx0)
    pltpu.async_copy(y.at[sidx0.at[0]], rows0, sem0)
    pltpu.make_async_copy(y.at[pl.ds(s * RPS, RPS)],
                          accsh.at[pl.ds(s * RPS, RPS)], isem).wait()
    plsc.subcore_barrier()

    for st in range(STAGES):
        sidx, didx = (sidx0, didx0) if st % 2 == 0 else (sidx1, didx1)
        nsidx, ndidx = (sidx1, didx1) if st % 2 == 0 else (sidx0, didx0)
        if st < STAGES - 1:
            pltpu.async_copy(src4d.at[w, st + 1], nsidx, xsem)
            pltpu.async_copy(dst4d.at[w, st + 1], ndidx, xsem)

        def pair(p, _):
            j0 = 2 * p
            pltpu.make_async_copy(y.at[sidx.at[j0]], rows0, sem0).wait()
            pltpu.async_copy(y.at[sidx.at[j0 + 1]], rows1, sem1)
            pltpu.sync_copy(rows0, accsh.at[didx.at[j0]], add=True)
            pltpu.async_copy(y.at[sidx.at[j0 + 2]], rows0, sem0)
            pltpu.make_async_copy(y.at[sidx.at[j0 + 1]], rows1, sem1).wait()
            pltpu.sync_copy(rows1, accsh.at[didx.at[j0 + 1]], add=True)
            return 0
        lax.fori_loop(0, PAIRS - 1, pair, 0)

        jl = CPS - 2
        pltpu.make_async_copy(y.at[sidx.at[jl]], rows0, sem0).wait()
        pltpu.async_copy(y.at[sidx.at[jl + 1]], rows1, sem1)
        pltpu.sync_copy(rows0, accsh.at[didx.at[jl]], add=True)
        if st < STAGES - 1:
            pltpu.make_async_copy(src4d.at[w, st + 1], nsidx, xsem).wait()
            pltpu.make_async_copy(dst4d.at[w, st + 1], ndidx, xsem).wait()
            pltpu.async_copy(y.at[nsidx.at[0]], rows0, sem0)
        pltpu.make_async_copy(y.at[sidx.at[jl + 1]], rows1, sem1).wait()
        pltpu.sync_copy(rows1, accsh.at[didx.at[jl + 1]], add=True)

    plsc.subcore_barrier()
    pltpu.sync_copy(accsh.at[pl.ds(s * RPS, RPS)],
                    out.at[c, pl.ds(s * RPS, RPS)])


_scat_call = pl.kernel(
    _scat_body,
    out_type=jax.ShapeDtypeStruct((NC, AR, D), jnp.float32),
    mesh=_mesh,
    scratch_types=[
        pltpu.VMEM((CPS, CH), jnp.int32),
        pltpu.VMEM((CPS, CH), jnp.int32),
        pltpu.VMEM((CPS, CH), jnp.int32),
        pltpu.VMEM((CPS, CH), jnp.int32),
        pltpu.VMEM((CH, D), jnp.float32),
        pltpu.VMEM((CH, D), jnp.float32),
        pltpu.VMEM_SHARED((AR, D), jnp.float32),
        pltpu.SemaphoreType.DMA,
        pltpu.SemaphoreType.DMA,
        pltpu.SemaphoreType.DMA,
        pltpu.SemaphoreType.DMA,
    ],
)


def _prep1_body(degp, x, w1, dis_ref, y1_ref):
    dd = degp[...]
    comb = dd[0] + dd[1]
    rid = lax.broadcasted_iota(jnp.int32, (8, NP), 0)
    comb = jnp.where(rid == 0, comb, 0.0)
    ones8 = jnp.ones((8, 8), jnp.bfloat16)
    degt = lax.dot_general(comb.astype(jnp.bfloat16), ones8,
                           (((0,), (0,)), ((), ())),
                           preferred_element_type=jnp.float32)
    deg = degt[0:N, 0:1] + 1.0
    dis = lax.rsqrt(deg)
    dis_ref[...] = dis
    xw = _dot_w(x[...], w1[...])
    y1_ref[0:N, :] = xw * dis
    y1_ref[N:NP, :] = jnp.zeros((NP - N, D), jnp.float32)


_prep1_call = pl.pallas_call(
    _prep1_body,
    out_shape=[
        jax.ShapeDtypeStruct((N, 1), jnp.float32),
        jax.ShapeDtypeStruct((NP, D), jnp.float32),
    ],
)


def _mid_body(acc, y1, dis, b1, w2, y2_ref):
    comb = acc[0, 0:N, :] + acc[1, 0:N, :] - y1[0:N, :]
    h1 = jnp.maximum(comb * dis[...] + b1[...], 0.0)
    xw2 = _dot_w(h1, w2[...])
    y2_ref[0:N, :] = xw2 * dis[...]
    y2_ref[N:NP, :] = jnp.zeros((NP - N, D), jnp.float32)


_mid_call = pl.pallas_call(
    _mid_body,
    out_shape=jax.ShapeDtypeStruct((NP, D), jnp.float32),
)


def _final_body(acc, y2, dis, b2, batch, wih, bih, bhh, wlin, blin, out_ref):
    h2 = ((acc[0, 0:N, :] + acc[1, 0:N, :] - y2[0:N, :]) * dis[...]
          + b2[...])
    b = batch[...]
    gids = lax.broadcasted_iota(jnp.int32, (1, G), 1)
    oh = (b == gids).astype(jnp.float32)
    h2_hi = h2.astype(jnp.bfloat16)
    h2_lo = (h2 - h2_hi.astype(jnp.float32)).astype(jnp.bfloat16)
    ohb = oh.astype(jnp.bfloat16)
    dn = (((0,), (0,)), ((), ()))
    psum = (lax.dot_general(ohb, h2_hi, dn, preferred_element_type=jnp.float32)
            + lax.dot_general(ohb, h2_lo, dn, preferred_element_type=jnp.float32))
    ones = jnp.ones((N, 1), jnp.bfloat16)
    cnt = lax.dot_general(ohb, ones, dn, preferred_element_type=jnp.float32)
    pooled = psum / jnp.maximum(cnt, 1.0)
    gi = _dot_w(pooled, wih[...]) + bih[...]
    gh = bhh[...]
    r = jax.nn.sigmoid(gi[:, :H] + gh[:, :H])
    z = jax.nn.sigmoid(gi[:, H:2 * H] + gh[:, H:2 * H])
    nn = jnp.tanh(gi[:, 2 * H:] + r * gh[:, 2 * H:])
    g = jnp.maximum((1.0 - z) * nn, 0.0)
    mu = jnp.mean(g, axis=-1, keepdims=True)
    var = jnp.mean((g - mu) ** 2, axis=-1, keepdims=True)
    gn = (g - mu) / jnp.sqrt(var + 1e-5)
    gnb = gn.astype(jnp.bfloat16).astype(jnp.float32)
    wlb = wlin[...].astype(jnp.bfloat16).astype(jnp.float32)
    out_ref[...] = jnp.sum(gnb * wlb, axis=1, keepdims=True) + blin[...]


_final_call = pl.pallas_call(
    _final_body,
    out_shape=jax.ShapeDtypeStruct((G, 1), jnp.float32),
)


def kernel(x, edge_index, batch, W1, b1, W2, b2, W_ih, W_hh, b_ih, b_hh,
           W_lin, b_lin):
    padv = PAD_DST + (jnp.arange(EP - E, dtype=jnp.int32) % (AR - N))
    src4d = jnp.concatenate([edge_index[0], padv]).reshape(
        NW, STAGES, CPS, CH)
    dst4d = jnp.concatenate([edge_index[1], padv]).reshape(
        NW, STAGES, CPS, CH)
    deg_out = _deg_call(dst4d)
    dis, y1 = _prep1_call(deg_out, x, W1)
    acc1 = _scat_call(y1, src4d, dst4d)
    y2 = _mid_call(acc1, y1, dis, b1.reshape(1, H), W2)
    acc2 = _scat_call(y2, src4d, dst4d)
    out = _final_call(acc2, y2, dis, b2.reshape(1, H),
                      batch.reshape(N, 1), W_ih, b_ih.reshape(1, 3 * H),
                      b_hh.reshape(1, 3 * H), W_lin, b_lin.reshape(1, 1))
    return out
